# baseline (speedup 1.0000x reference)
import functools

import jax
import jax.numpy as jnp
from jax import lax
from jax.experimental import pallas as pl
from jax.experimental.pallas import tpu as pltpu
from jax.experimental.pallas import tpu_sc as plsc

NUM_CORES = 2
NUM_SUBCORES = 16
LANES = 16
NW = NUM_CORES * NUM_SUBCORES

BATCH = 16384
EMBED_DIM = 64
B_PER_W = BATCH // NW
IDX_CHUNK = 128
N_CHUNKS = B_PER_W // IDX_CHUNK


def _body(cidx_hbm, aidx_hbm, ctable_hbm, atable_hbm, out_hbm,
          cidx_v, aidx_v, crows_v, arows_v, out_v, sem):
    wid = lax.axis_index("s") * NUM_CORES + lax.axis_index("c")
    base = wid * N_CHUNKS

    pltpu.sync_copy(cidx_hbm.at[pl.ds(base, N_CHUNKS)], cidx_v)
    pltpu.sync_copy(aidx_hbm.at[pl.ds(base, N_CHUNKS)], aidx_v)

    copies = []
    for k in range(N_CHUNKS):
        copies.append(pltpu.async_copy(
            ctable_hbm.at[cidx_v.at[k]],
            crows_v.at[pl.ds(k * IDX_CHUNK, IDX_CHUNK)], sem))
        copies.append(pltpu.async_copy(
            atable_hbm.at[aidx_v.at[k]],
            arows_v.at[pl.ds(k * IDX_CHUNK, IDX_CHUNK)], sem))
    for c in copies:
        c.wait()

    lane = lax.iota(jnp.int32, LANES)
    perms = [(lane ^ m).reshape(LANES, 1) for m in (8, 4, 2, 1)]
    dnums = lax.GatherDimensionNumbers(
        offset_dims=(), collapsed_slice_dims=(0,), start_index_map=(0,))

    def shuffle(x, p):
        return lax.gather(x, p, dnums, slice_sizes=(1,),
                          mode=lax.GatherScatterMode.PROMISE_IN_BOUNDS)

    def group_body(g, carry):
        out_vec = jnp.zeros((LANES,), jnp.float32)
        for l in range(LANES):
            r = g * LANES + l
            acc = crows_v[r, pl.ds(0, LANES)] * arows_v[r, pl.ds(0, LANES)]
            for j in range(1, EMBED_DIM // LANES):
                acc = acc + (crows_v[r, pl.ds(j * LANES, LANES)]
                             * arows_v[r, pl.ds(j * LANES, LANES)])
            for p in perms:
                acc = acc + shuffle(acc, p)
            out_vec = jnp.where(lane == l, acc, out_vec)
        out_v[pl.ds(g * LANES, LANES)] = out_vec
        return carry

    lax.fori_loop(0, B_PER_W // LANES, group_body, 0)

    pltpu.sync_copy(out_v, out_hbm.at[pl.ds(wid * B_PER_W, B_PER_W)])


@jax.jit
def kernel(customer, article, customer_table, article_table):
    mesh = plsc.VectorSubcoreMesh(core_axis_name="c", subcore_axis_name="s")
    run = pl.kernel(
        _body,
        out_type=jax.ShapeDtypeStruct((BATCH,), jnp.float32),
        mesh=mesh,
        compiler_params=pltpu.CompilerParams(use_tc_tiling_on_sc=False),
        scratch_types=[
            pltpu.VMEM((N_CHUNKS, IDX_CHUNK), jnp.int32),
            pltpu.VMEM((N_CHUNKS, IDX_CHUNK), jnp.int32),
            pltpu.VMEM((B_PER_W, EMBED_DIM), jnp.float32),
            pltpu.VMEM((B_PER_W, EMBED_DIM), jnp.float32),
            pltpu.VMEM((B_PER_W,), jnp.float32),
            pltpu.SemaphoreType.DMA,
        ],
    )
    cidx = customer.reshape(NW * N_CHUNKS, IDX_CHUNK)
    aidx = article.reshape(NW * N_CHUNKS, IDX_CHUNK)
    return run(cidx, aidx, customer_table, article_table)


# speedup vs baseline: 1.5969x; 1.5969x over previous
import jax
import jax.numpy as jnp
from jax import lax
from jax.experimental import pallas as pl
from jax.experimental.pallas import tpu as pltpu
from jax.experimental.pallas import tpu_sc as plsc

NUM_CORES = 2
NUM_SUBCORES = 16
LANES = 16
NW = NUM_CORES * NUM_SUBCORES

BATCH = 16384
EMBED_DIM = 64
B_PER_W = BATCH // NW
CHUNK = 32
N_CHUNKS = B_PER_W // CHUNK
GROUPS = CHUNK // LANES


def _body(cidx_hbm, aidx_hbm, ctable_hbm, atable_hbm, out_hbm,
          cidx_v, aidx_v, cbuf_v, abuf_v, out_v, sem):
    wid = lax.axis_index("s") * NUM_CORES + lax.axis_index("c")
    base = wid * N_CHUNKS

    pltpu.sync_copy(cidx_hbm.at[pl.ds(base, N_CHUNKS)], cidx_v)
    pltpu.sync_copy(aidx_hbm.at[pl.ds(base, N_CHUNKS)], aidx_v)

    lane = lax.iota(jnp.int32, LANES)
    perms = [(lane ^ m).reshape(LANES, 1) for m in (8, 4, 2, 1)]
    dnums = lax.GatherDimensionNumbers(
        offset_dims=(), collapsed_slice_dims=(0,), start_index_map=(0,))

    def shuffle(x, p):
        return lax.gather(x, p, dnums, slice_sizes=(1,),
                          mode=lax.GatherScatterMode.PROMISE_IN_BOUNDS)

    def chunk_body(k, carry):
        copies = []
        for g in range(GROUPS):
            civ = cidx_v[k, pl.ds(g * LANES, LANES)]
            aiv = aidx_v[k, pl.ds(g * LANES, LANES)]
            for l in range(LANES):
                j = g * LANES + l
                copies.append(pltpu.async_copy(
                    ctable_hbm.at[pl.ds(civ[l], 1)],
                    cbuf_v.at[pl.ds(j, 1)], sem))
                copies.append(pltpu.async_copy(
                    atable_hbm.at[pl.ds(aiv[l], 1)],
                    abuf_v.at[pl.ds(j, 1)], sem))
        for c in copies:
            c.wait()

        for g in range(GROUPS):
            out_vec = jnp.zeros((LANES,), jnp.float32)
            for l in range(LANES):
                j = g * LANES + l
                acc = (cbuf_v[j, pl.ds(0, LANES)]
                       * abuf_v[j, pl.ds(0, LANES)])
                for d in range(1, EMBED_DIM // LANES):
                    acc = acc + (cbuf_v[j, pl.ds(d * LANES, LANES)]
                                 * abuf_v[j, pl.ds(d * LANES, LANES)])
                for p in perms:
                    acc = acc + shuffle(acc, p)
                out_vec = jnp.where(lane == l, acc, out_vec)
            out_v[pl.ds(k * CHUNK + g * LANES, LANES)] = out_vec
        return carry

    lax.fori_loop(0, N_CHUNKS, chunk_body, 0)

    pltpu.sync_copy(out_v, out_hbm.at[pl.ds(wid * B_PER_W, B_PER_W)])


@jax.jit
def kernel(customer, article, customer_table, article_table):
    mesh = plsc.VectorSubcoreMesh(core_axis_name="c", subcore_axis_name="s")
    run = pl.kernel(
        _body,
        out_type=jax.ShapeDtypeStruct((BATCH,), jnp.float32),
        mesh=mesh,
        scratch_types=[
            pltpu.VMEM((N_CHUNKS, CHUNK), jnp.int32),
            pltpu.VMEM((N_CHUNKS, CHUNK), jnp.int32),
            pltpu.VMEM((CHUNK, EMBED_DIM), jnp.float32),
            pltpu.VMEM((CHUNK, EMBED_DIM), jnp.float32),
            pltpu.VMEM((B_PER_W,), jnp.float32),
            pltpu.SemaphoreType.DMA,
        ],
    )
    cidx = customer.reshape(NW * N_CHUNKS, CHUNK)
    aidx = article.reshape(NW * N_CHUNKS, CHUNK)
    return run(cidx, aidx, customer_table, article_table)


# speedup vs baseline: 1.5971x; 1.0001x over previous
import jax
import jax.numpy as jnp
from jax import lax
from jax.experimental import pallas as pl
from jax.experimental.pallas import tpu as pltpu
from jax.experimental.pallas import tpu_sc as plsc

NUM_CORES = 2
NUM_SUBCORES = 16
LANES = 16
NW = NUM_CORES * NUM_SUBCORES

BATCH = 16384
EMBED_DIM = 64
B_PER_W = BATCH // NW
CHUNK = 32
N_CHUNKS = B_PER_W // CHUNK
GROUPS = CHUNK // LANES


def _body(cidx_hbm, aidx_hbm, ctable_hbm, atable_hbm, out_hbm,
          cidx_v, aidx_v, cbuf_v, abuf_v, out_v, sem, *sems):
    wid = lax.axis_index("s") * NUM_CORES + lax.axis_index("c")
    base = wid * N_CHUNKS

    pltpu.sync_copy(cidx_hbm.at[pl.ds(base, N_CHUNKS)], cidx_v)
    pltpu.sync_copy(aidx_hbm.at[pl.ds(base, N_CHUNKS)], aidx_v)

    lane = lax.iota(jnp.int32, LANES)
    perms = [(lane ^ m).reshape(LANES, 1) for m in (8, 4, 2, 1)]
    dnums = lax.GatherDimensionNumbers(
        offset_dims=(), collapsed_slice_dims=(0,), start_index_map=(0,))

    def shuffle(x, p):
        return lax.gather(x, p, dnums, slice_sizes=(1,),
                          mode=lax.GatherScatterMode.PROMISE_IN_BOUNDS)

    def chunk_body(k, carry):
        copies = []
        for g in range(GROUPS):
            civ = cidx_v[k, pl.ds(g * LANES, LANES)]
            aiv = aidx_v[k, pl.ds(g * LANES, LANES)]
            for l in range(LANES):
                j = g * LANES + l
                copies.append(pltpu.async_copy(
                    ctable_hbm.at[pl.ds(civ[l], 1)],
                    cbuf_v.at[pl.ds(j, 1)], sems[(2 * j) % len(sems)]))
                copies.append(pltpu.async_copy(
                    atable_hbm.at[pl.ds(aiv[l], 1)],
                    abuf_v.at[pl.ds(j, 1)], sems[(2 * j + 1) % len(sems)]))
        for c in copies:
            c.wait()

        for g in range(GROUPS):
            out_vec = jnp.zeros((LANES,), jnp.float32)
            for l in range(LANES):
                j = g * LANES + l
                acc = (cbuf_v[j, pl.ds(0, LANES)]
                       * abuf_v[j, pl.ds(0, LANES)])
                for d in range(1, EMBED_DIM // LANES):
                    acc = acc + (cbuf_v[j, pl.ds(d * LANES, LANES)]
                                 * abuf_v[j, pl.ds(d * LANES, LANES)])
                for p in perms:
                    acc = acc + shuffle(acc, p)
                out_vec = jnp.where(lane == l, acc, out_vec)
            out_v[pl.ds(k * CHUNK + g * LANES, LANES)] = out_vec
        return carry

    lax.fori_loop(0, N_CHUNKS, chunk_body, 0)

    pltpu.sync_copy(out_v, out_hbm.at[pl.ds(wid * B_PER_W, B_PER_W)])


@jax.jit
def kernel(customer, article, customer_table, article_table):
    mesh = plsc.VectorSubcoreMesh(core_axis_name="c", subcore_axis_name="s")
    run = pl.kernel(
        _body,
        out_type=jax.ShapeDtypeStruct((BATCH,), jnp.float32),
        mesh=mesh,
        scratch_types=[
            pltpu.VMEM((N_CHUNKS, CHUNK), jnp.int32),
            pltpu.VMEM((N_CHUNKS, CHUNK), jnp.int32),
            pltpu.VMEM((CHUNK, EMBED_DIM), jnp.float32),
            pltpu.VMEM((CHUNK, EMBED_DIM), jnp.float32),
            pltpu.VMEM((B_PER_W,), jnp.float32),
            pltpu.SemaphoreType.DMA,
        ] + [pltpu.SemaphoreType.DMA] * 8,
    )
    cidx = customer.reshape(NW * N_CHUNKS, CHUNK)
    aidx = article.reshape(NW * N_CHUNKS, CHUNK)
    return run(cidx, aidx, customer_table, article_table)
